# baseline (speedup 1.0000x reference)
import functools

import jax
import jax.numpy as jnp
from jax import lax
from jax.experimental import pallas as pl
from jax.experimental.pallas import tpu as pltpu
from jax.experimental.pallas import tpu_sc as plsc

N = 10000
F = 128
E = 320000
NQ = 50
NR = 64

NC, NS = 2, 16
NW = NC * NS
CHUNK = 128
NCH = 80
NBUF = 2
ND = 4
EPAD = NW * NCH * CHUNK
NPAD = NW * 320
TROW = 632
NACC = NS * TROW
JUNK = N + 16


def _emb_body(xp_hbm, emb_hbm, out_hbm, x_v, rows_v, sem):
    c = lax.axis_index("c")
    s = lax.axis_index("s")
    wid = s * NC + c
    pltpu.sync_copy(xp_hbm.at[wid], x_v)
    for k in range(4):
        pltpu.async_copy(emb_hbm.at[x_v.at[pl.ds(k * 80, 80)]], rows_v, sem).wait()
        pltpu.sync_copy(rows_v, out_hbm.at[pl.ds(wid * 320 + k * 80, 80)])


def _deg_body(dstp_hbm, out_hbm, dst_v, zbuf_v, ones_v, deg_sp):
    c = lax.axis_index("c")
    s = lax.axis_index("s")
    wid = s * NC + c
    for j in range(640 // 16):
        zbuf_v[pl.ds(j * 16, 16)] = jnp.zeros((16,), jnp.float32)
    for j in range(CHUNK // 16):
        ones_v[pl.ds(j * 16, 16)] = jnp.full((16,), 1.0, jnp.float32)
    pltpu.sync_copy(zbuf_v, deg_sp.at[pl.ds(s * 640, 640)])
    plsc.subcore_barrier()
    pltpu.sync_copy(dstp_hbm.at[wid], dst_v)

    def chunk(i, carry):
        pltpu.sync_copy(ones_v, deg_sp.at[dst_v.at[i]], add=True)
        return carry

    lax.fori_loop(0, NCH, chunk, 0)
    plsc.subcore_barrier()
    pltpu.sync_copy(deg_sp.at[pl.ds(s * 640, 640)], out_hbm.at[c, pl.ds(s * 640, 640)])


def _scatter_body(g_hbm, idxp_hbm, out_hbm, idx_v, rows_v, acc_sp,
                  sem0, sem1, isem0, isem1, isem2, isem3):
    c = lax.axis_index("c")
    s = lax.axis_index("s")
    wid = s * NC + c
    sems = (sem0, sem1)
    isems = (isem0, isem1, isem2, isem3)

    def zrow(i, carry):
        for j in range(F // 16):
            rows_v[0, i, pl.ds(j * 16, 16)] = jnp.zeros((16,), jnp.float32)
        return carry

    lax.fori_loop(0, CHUNK, zrow, 0)
    for k in range(TROW // CHUNK):
        pltpu.sync_copy(rows_v.at[0], acc_sp.at[pl.ds(s * TROW + k * CHUNK, CHUNK)])
    rem = TROW % CHUNK
    pltpu.sync_copy(rows_v.at[0, pl.ds(0, rem)],
                    acc_sp.at[pl.ds(s * TROW + TROW - rem, rem)])
    plsc.subcore_barrier()

    ih = [pltpu.async_copy(idxp_hbm.at[wid, d], idx_v.at[d], isems[d])
          for d in range(ND)]
    rh = [None, None]
    for b in range(NBUF):
        ih[b].wait()
        rh[b] = pltpu.async_copy(g_hbm.at[idx_v.at[b, 0]], rows_v.at[b], sems[b])
    for ch in range(NCH):
        b = ch % NBUF
        d = ch % ND
        rh[b].wait()
        pltpu.sync_copy(rows_v.at[b], acc_sp.at[idx_v.at[d, 1]], add=True)
        nxt = ch + NBUF
        if nxt < NCH:
            dn = nxt % ND
            ih[dn].wait()
            rh[b] = pltpu.async_copy(
                g_hbm.at[idx_v.at[dn, 0]], rows_v.at[b], sems[b])
        if ch + ND < NCH:
            ih[d] = pltpu.async_copy(
                idxp_hbm.at[wid, ch + ND], idx_v.at[d], isems[d])
    plsc.subcore_barrier()
    pltpu.sync_copy(acc_sp.at[pl.ds(s * TROW, TROW)],
                    out_hbm.at[c, pl.ds(s * TROW, TROW)])


@functools.lru_cache(maxsize=None)
def _sc_kernels():
    mesh = plsc.VectorSubcoreMesh(
        core_axis_name="c", subcore_axis_name="s",
        num_cores=NC, num_subcores=NS)
    emb = pl.kernel(
        _emb_body,
        out_type=jax.ShapeDtypeStruct((NPAD, F), jnp.float32),
        mesh=mesh,
        scratch_types=[
            pltpu.VMEM((320,), jnp.int32),
            pltpu.VMEM((80, F), jnp.float32),
            pltpu.SemaphoreType.DMA,
        ])
    deg = pl.kernel(
        _deg_body,
        out_type=jax.ShapeDtypeStruct((NC, NPAD), jnp.float32),
        mesh=mesh,
        scratch_types=[
            pltpu.VMEM((NCH, CHUNK), jnp.int32),
            pltpu.VMEM((640,), jnp.float32),
            pltpu.VMEM((CHUNK,), jnp.float32),
            pltpu.VMEM_SHARED((NPAD,), jnp.float32),
        ])
    scat = pl.kernel(
        _scatter_body,
        out_type=jax.ShapeDtypeStruct((NC, NACC, F), jnp.float32),
        mesh=mesh,
        scratch_types=[
            pltpu.VMEM((ND, 2, CHUNK), jnp.int32),
            pltpu.VMEM((NBUF, CHUNK, F), jnp.float32),
            pltpu.VMEM_SHARED((NACC, F), jnp.float32),
            pltpu.SemaphoreType.DMA,
            pltpu.SemaphoreType.DMA,
            pltpu.SemaphoreType.DMA,
            pltpu.SemaphoreType.DMA,
            pltpu.SemaphoreType.DMA,
            pltpu.SemaphoreType.DMA,
        ])
    return emb, deg, scat


def _emb_gather(xp, emb):
    return _sc_kernels()[0](xp, emb)


def _deg_hist(dstp):
    return _sc_kernels()[1](dstp)


def _edge_scatter(g, idxp):
    return _sc_kernels()[2](g, idxp)


BR = 2000


def _g0_body(h_ref, w_ref, d0_ref, d1_ref, g_ref, dinv_ref):
    dinv = lax.rsqrt(d0_ref[...] + d1_ref[...] + 1.0)
    dinv_ref[...] = dinv
    g_ref[...] = jnp.dot(h_ref[...], w_ref[...],
                         preferred_element_type=jnp.float32) * dinv


def _g0(h, w, d0, d1):
    return pl.pallas_call(
        _g0_body,
        grid=(N // BR,),
        in_specs=[
            pl.BlockSpec((BR, F), lambda i: (i, 0)),
            pl.BlockSpec((F, F), lambda i: (0, 0)),
            pl.BlockSpec((BR, 1), lambda i: (i, 0)),
            pl.BlockSpec((BR, 1), lambda i: (i, 0)),
        ],
        out_specs=[
            pl.BlockSpec((BR, F), lambda i: (i, 0)),
            pl.BlockSpec((BR, 1), lambda i: (i, 0)),
        ],
        out_shape=[
            jax.ShapeDtypeStruct((N, F), jnp.float32),
            jax.ShapeDtypeStruct((N, 1), jnp.float32),
        ],
    )(h, w, d0, d1)


def _step_body(h_ref, g_ref, a0_ref, a1_ref, dinv_ref, b_ref, w_ref,
               hn_ref, gn_ref):
    dinv = dinv_ref[...]
    o = dinv * (a0_ref[...] + a1_ref[...] + g_ref[...]) + b_ref[...]
    o = jnp.where(o > 0, o, 0.01 * o)
    hn = o + h_ref[...]
    hn_ref[...] = hn
    gn_ref[...] = jnp.dot(hn, w_ref[...],
                          preferred_element_type=jnp.float32) * dinv


def _step(h, g, a0, a1, dinv, b, w):
    return pl.pallas_call(
        _step_body,
        grid=(N // BR,),
        in_specs=[
            pl.BlockSpec((BR, F), lambda i: (i, 0)),
            pl.BlockSpec((BR, F), lambda i: (i, 0)),
            pl.BlockSpec((BR, F), lambda i: (i, 0)),
            pl.BlockSpec((BR, F), lambda i: (i, 0)),
            pl.BlockSpec((BR, 1), lambda i: (i, 0)),
            pl.BlockSpec((1, F), lambda i: (0, 0)),
            pl.BlockSpec((F, F), lambda i: (0, 0)),
        ],
        out_specs=[
            pl.BlockSpec((BR, F), lambda i: (i, 0)),
            pl.BlockSpec((BR, F), lambda i: (i, 0)),
        ],
        out_shape=[
            jax.ShapeDtypeStruct((N, F), jnp.float32),
            jax.ShapeDtypeStruct((N, F), jnp.float32),
        ],
    )(h, g, a0, a1, dinv, b, w)


def _final_body(h_ref, g_ref, a0_ref, a1_ref, dinv_ref, b_ref, hn_ref):
    o = dinv_ref[...] * (a0_ref[...] + a1_ref[...] + g_ref[...]) + b_ref[...]
    hn_ref[...] = o + h_ref[...]


def _final(h, g, a0, a1, dinv, b):
    return pl.pallas_call(
        _final_body,
        out_shape=jax.ShapeDtypeStruct((NR, F), jnp.float32),
    )(h, g, a0, a1, dinv, b)


def _lrelu(t):
    return jnp.where(t > 0, t, 0.01 * t)


def _mlp_body(h_ref, w0_ref, b0_ref, w1_ref, b1_ref, w2_ref, b2_ref,
              w3_ref, b3_ref, w4_ref, b4_ref, w5_ref, b5_ref, out_ref):
    h = h_ref[...]
    a = jnp.dot(h, w0_ref[0:F, :], preferred_element_type=jnp.float32)
    b = jnp.dot(h, w0_ref[F:2 * F, :], preferred_element_type=jnp.float32)

    def tail(z):
        z = _lrelu(z + b0_ref[...])
        z = _lrelu(jnp.dot(z, w1_ref[...], preferred_element_type=jnp.float32) + b1_ref[...])
        z = _lrelu(jnp.dot(z, w2_ref[...], preferred_element_type=jnp.float32) + b2_ref[...])
        z = _lrelu(jnp.dot(z, w3_ref[...], preferred_element_type=jnp.float32) + b3_ref[...])
        z = _lrelu(jnp.dot(z, w4_ref[...], preferred_element_type=jnp.float32) + b4_ref[...])
        return jnp.dot(z, w5_ref[...], preferred_element_type=jnp.float32) + b5_ref[...]

    z1 = (a.reshape(NR, 1, 2 * F) + b.reshape(1, NR, 2 * F)).reshape(NR * NR, 2 * F)
    z2 = (b.reshape(NR, 1, 2 * F) + a.reshape(1, NR, 2 * F)).reshape(NR * NR, 2 * F)
    out_ref[...] = (tail(z1) + tail(z2)) * 0.5


def _mlp(h64, ws):
    return pl.pallas_call(
        _mlp_body,
        out_shape=jax.ShapeDtypeStruct((NR * NR, F), jnp.float32),
    )(h64, *ws)


def kernel(x, edge_index, emb, gcnW, gcnb,
           m0W, m0b, m1W, m1b, m2W, m2b, m3W, m3b, m4W, m4b, m5W, m5b):
    src = edge_index[0].astype(jnp.int32)
    dst = edge_index[1].astype(jnp.int32)
    srcp = jnp.concatenate(
        [src, jnp.zeros((EPAD - E,), jnp.int32)]).reshape(NW, NCH, CHUNK)
    dstp = jnp.concatenate(
        [dst, jnp.full((EPAD - E,), JUNK, jnp.int32)]).reshape(NW, NCH, CHUNK)
    idxp = jnp.stack([srcp, dstp], axis=2)
    xp = jnp.concatenate(
        [x.astype(jnp.int32), jnp.zeros((NPAD - N,), jnp.int32)]).reshape(NW, 320)

    h0 = _emb_gather(xp, emb)[:N]
    degs = _deg_hist(dstp)
    d0 = degs[0, :N, None]
    d1 = degs[1, :N, None]

    h = h0
    g, dinv = _g0(h0, gcnW[0], d0, d1)
    for i in range(4):
        acc = _edge_scatter(g, idxp)
        h, g = _step(h, g, acc[0, :N], acc[1, :N], dinv, gcnb[i][None], gcnW[i + 1])
    acc = _edge_scatter(g, idxp)
    h64 = _final(h[:NR], g[:NR], acc[0, :NR], acc[1, :NR], dinv[:NR], gcnb[4][None])

    m5Wp = jnp.pad(m5W, ((0, 0), (0, F - 1)))
    m5bp = jnp.pad(m5b, (0, F - 1))
    ws = (m0W, m0b[None], m1W, m1b[None], m2W, m2b[None],
          m3W, m3b[None], m4W, m4b[None], m5Wp, m5bp[None])
    zz = _mlp(h64, ws)
    z = zz[:, 0].reshape(NR, NR)[:NQ, :NQ]
    return z.reshape(1, NQ * NQ)


# speedup vs baseline: 1.6027x; 1.6027x over previous
import functools

import jax
import jax.numpy as jnp
from jax import lax
from jax.experimental import pallas as pl
from jax.experimental.pallas import tpu as pltpu
from jax.experimental.pallas import tpu_sc as plsc

N = 10000
F = 128
E = 320000
NQ = 50
NR = 64

NC, NS = 2, 16
NW = NC * NS
CHUNK = 128
NCH = 79
EPAD = NW * NCH * CHUNK
NPAD = NW * 320
TROW = 632
NACC = NS * TROW
JUNK = N + 16


def _emb_body(xp_hbm, emb_hbm, out_hbm, x_v, rows_v, sem):
    c = lax.axis_index("c")
    s = lax.axis_index("s")
    wid = s * NC + c
    pltpu.sync_copy(xp_hbm.at[wid], x_v)
    for k in range(4):
        pltpu.async_copy(emb_hbm.at[x_v.at[pl.ds(k * 80, 80)]], rows_v, sem).wait()
        pltpu.sync_copy(rows_v, out_hbm.at[pl.ds(wid * 320 + k * 80, 80)])


def _deg_body(dstp_hbm, out_hbm, dst_v, zbuf_v, ones_v, deg_sp):
    c = lax.axis_index("c")
    s = lax.axis_index("s")
    wid = s * NC + c
    for j in range(640 // 16):
        zbuf_v[pl.ds(j * 16, 16)] = jnp.zeros((16,), jnp.float32)
    for j in range(CHUNK // 16):
        ones_v[pl.ds(j * 16, 16)] = jnp.full((16,), 1.0, jnp.float32)
    pltpu.sync_copy(zbuf_v, deg_sp.at[pl.ds(s * 640, 640)])
    plsc.subcore_barrier()
    pltpu.sync_copy(dstp_hbm.at[wid], dst_v)

    def chunk(i, carry):
        pltpu.sync_copy(ones_v, deg_sp.at[dst_v.at[i]], add=True)
        return carry

    lax.fori_loop(0, NCH, chunk, 0)
    plsc.subcore_barrier()
    pltpu.sync_copy(deg_sp.at[pl.ds(s * 640, 640)], out_hbm.at[c, pl.ds(s * 640, 640)])


def _scatter_body(g_hbm, srcp_hbm, dstp_hbm, out_hbm, src_v, dst_v, rows_v, acc_sp, sem):
    c = lax.axis_index("c")
    s = lax.axis_index("s")
    wid = s * NC + c

    def zrow(i, carry):
        for j in range(F // 16):
            rows_v[i, pl.ds(j * 16, 16)] = jnp.zeros((16,), jnp.float32)
        return carry

    lax.fori_loop(0, CHUNK, zrow, 0)
    for k in range(TROW // CHUNK):
        pltpu.sync_copy(rows_v, acc_sp.at[pl.ds(s * TROW + k * CHUNK, CHUNK)])
    rem = TROW % CHUNK
    pltpu.sync_copy(rows_v.at[pl.ds(0, rem)],
                    acc_sp.at[pl.ds(s * TROW + TROW - rem, rem)])
    plsc.subcore_barrier()

    pltpu.sync_copy(srcp_hbm.at[wid], src_v)
    pltpu.sync_copy(dstp_hbm.at[wid], dst_v)

    def chunk(i, carry):
        pltpu.async_copy(g_hbm.at[src_v.at[i]], rows_v, sem).wait()
        pltpu.sync_copy(rows_v, acc_sp.at[dst_v.at[i]], add=True)
        return carry

    lax.fori_loop(0, NCH, chunk, 0)
    plsc.subcore_barrier()
    pltpu.sync_copy(acc_sp.at[pl.ds(s * TROW, TROW)],
                    out_hbm.at[c, pl.ds(s * TROW, TROW)])


@functools.lru_cache(maxsize=None)
def _sc_kernels():
    mesh = plsc.VectorSubcoreMesh(
        core_axis_name="c", subcore_axis_name="s",
        num_cores=NC, num_subcores=NS)
    emb = pl.kernel(
        _emb_body,
        out_type=jax.ShapeDtypeStruct((NPAD, F), jnp.float32),
        mesh=mesh,
        scratch_types=[
            pltpu.VMEM((320,), jnp.int32),
            pltpu.VMEM((80, F), jnp.float32),
            pltpu.SemaphoreType.DMA,
        ])
    deg = pl.kernel(
        _deg_body,
        out_type=jax.ShapeDtypeStruct((NC, NPAD), jnp.float32),
        mesh=mesh,
        scratch_types=[
            pltpu.VMEM((NCH, CHUNK), jnp.int32),
            pltpu.VMEM((640,), jnp.float32),
            pltpu.VMEM((CHUNK,), jnp.float32),
            pltpu.VMEM_SHARED((NPAD,), jnp.float32),
        ])
    scat = pl.kernel(
        _scatter_body,
        out_type=jax.ShapeDtypeStruct((NC, NACC, F), jnp.float32),
        mesh=mesh,
        scratch_types=[
            pltpu.VMEM((NCH, CHUNK), jnp.int32),
            pltpu.VMEM((NCH, CHUNK), jnp.int32),
            pltpu.VMEM((CHUNK, F), jnp.float32),
            pltpu.VMEM_SHARED((NACC, F), jnp.float32),
            pltpu.SemaphoreType.DMA,
        ])
    return emb, deg, scat


def _emb_gather(xp, emb):
    return _sc_kernels()[0](xp, emb)


def _deg_hist(dstp):
    return _sc_kernels()[1](dstp)


def _edge_scatter(g, srcp, dstp):
    return _sc_kernels()[2](g, srcp, dstp)


BR = 2000


def _g0_body(h_ref, w_ref, d0_ref, d1_ref, g_ref, dinv_ref):
    dinv = lax.rsqrt(d0_ref[...] + d1_ref[...] + 1.0)
    dinv_ref[...] = dinv
    g_ref[...] = jnp.dot(h_ref[...], w_ref[...],
                         preferred_element_type=jnp.float32) * dinv


def _g0(h, w, d0, d1):
    return pl.pallas_call(
        _g0_body,
        grid=(N // BR,),
        in_specs=[
            pl.BlockSpec((BR, F), lambda i: (i, 0)),
            pl.BlockSpec((F, F), lambda i: (0, 0)),
            pl.BlockSpec((BR, 1), lambda i: (i, 0)),
            pl.BlockSpec((BR, 1), lambda i: (i, 0)),
        ],
        out_specs=[
            pl.BlockSpec((BR, F), lambda i: (i, 0)),
            pl.BlockSpec((BR, 1), lambda i: (i, 0)),
        ],
        out_shape=[
            jax.ShapeDtypeStruct((N, F), jnp.float32),
            jax.ShapeDtypeStruct((N, 1), jnp.float32),
        ],
    )(h, w, d0, d1)


def _step_body(h_ref, g_ref, a0_ref, a1_ref, dinv_ref, b_ref, w_ref,
               hn_ref, gn_ref):
    dinv = dinv_ref[...]
    o = dinv * (a0_ref[...] + a1_ref[...] + g_ref[...]) + b_ref[...]
    o = jnp.where(o > 0, o, 0.01 * o)
    hn = o + h_ref[...]
    hn_ref[...] = hn
    gn_ref[...] = jnp.dot(hn, w_ref[...],
                          preferred_element_type=jnp.float32) * dinv


def _step(h, g, a0, a1, dinv, b, w):
    return pl.pallas_call(
        _step_body,
        grid=(N // BR,),
        in_specs=[
            pl.BlockSpec((BR, F), lambda i: (i, 0)),
            pl.BlockSpec((BR, F), lambda i: (i, 0)),
            pl.BlockSpec((BR, F), lambda i: (i, 0)),
            pl.BlockSpec((BR, F), lambda i: (i, 0)),
            pl.BlockSpec((BR, 1), lambda i: (i, 0)),
            pl.BlockSpec((1, F), lambda i: (0, 0)),
            pl.BlockSpec((F, F), lambda i: (0, 0)),
        ],
        out_specs=[
            pl.BlockSpec((BR, F), lambda i: (i, 0)),
            pl.BlockSpec((BR, F), lambda i: (i, 0)),
        ],
        out_shape=[
            jax.ShapeDtypeStruct((N, F), jnp.float32),
            jax.ShapeDtypeStruct((N, F), jnp.float32),
        ],
    )(h, g, a0, a1, dinv, b, w)


def _final_body(h_ref, g_ref, a0_ref, a1_ref, dinv_ref, b_ref, hn_ref):
    o = dinv_ref[...] * (a0_ref[...] + a1_ref[...] + g_ref[...]) + b_ref[...]
    hn_ref[...] = o + h_ref[...]


def _final(h, g, a0, a1, dinv, b):
    return pl.pallas_call(
        _final_body,
        out_shape=jax.ShapeDtypeStruct((NR, F), jnp.float32),
    )(h, g, a0, a1, dinv, b)


def _lrelu(t):
    return jnp.where(t > 0, t, 0.01 * t)


def _mlp_body(h_ref, w0_ref, b0_ref, w1_ref, b1_ref, w2_ref, b2_ref,
              w3_ref, b3_ref, w4_ref, b4_ref, w5_ref, b5_ref, out_ref):
    h = h_ref[...]
    a = jnp.dot(h, w0_ref[0:F, :], preferred_element_type=jnp.float32)
    b = jnp.dot(h, w0_ref[F:2 * F, :], preferred_element_type=jnp.float32)

    def tail(z):
        z = _lrelu(z + b0_ref[...])
        z = _lrelu(jnp.dot(z, w1_ref[...], preferred_element_type=jnp.float32) + b1_ref[...])
        z = _lrelu(jnp.dot(z, w2_ref[...], preferred_element_type=jnp.float32) + b2_ref[...])
        z = _lrelu(jnp.dot(z, w3_ref[...], preferred_element_type=jnp.float32) + b3_ref[...])
        z = _lrelu(jnp.dot(z, w4_ref[...], preferred_element_type=jnp.float32) + b4_ref[...])
        return jnp.dot(z, w5_ref[...], preferred_element_type=jnp.float32) + b5_ref[...]

    z1 = (a.reshape(NR, 1, 2 * F) + b.reshape(1, NR, 2 * F)).reshape(NR * NR, 2 * F)
    z2 = (b.reshape(NR, 1, 2 * F) + a.reshape(1, NR, 2 * F)).reshape(NR * NR, 2 * F)
    out_ref[...] = (tail(z1) + tail(z2)) * 0.5


def _mlp(h64, ws):
    return pl.pallas_call(
        _mlp_body,
        out_shape=jax.ShapeDtypeStruct((NR * NR, F), jnp.float32),
    )(h64, *ws)


def kernel(x, edge_index, emb, gcnW, gcnb,
           m0W, m0b, m1W, m1b, m2W, m2b, m3W, m3b, m4W, m4b, m5W, m5b):
    src = edge_index[0].astype(jnp.int32)
    dst = edge_index[1].astype(jnp.int32)
    srcp = jnp.concatenate(
        [src, jnp.zeros((EPAD - E,), jnp.int32)]).reshape(NW, NCH, CHUNK)
    dstp = jnp.concatenate(
        [dst, jnp.full((EPAD - E,), JUNK, jnp.int32)]).reshape(NW, NCH, CHUNK)
    xp = jnp.concatenate(
        [x.astype(jnp.int32), jnp.zeros((NPAD - N,), jnp.int32)]).reshape(NW, 320)

    h0 = _emb_gather(xp, emb)[:N]
    degs = _deg_hist(dstp)
    d0 = degs[0, :N, None]
    d1 = degs[1, :N, None]

    h = h0
    g, dinv = _g0(h0, gcnW[0], d0, d1)
    for i in range(4):
        acc = _edge_scatter(g, srcp, dstp)
        h, g = _step(h, g, acc[0, :N], acc[1, :N], dinv, gcnb[i][None], gcnW[i + 1])
    acc = _edge_scatter(g, srcp, dstp)
    h64 = _final(h[:NR], g[:NR], acc[0, :NR], acc[1, :NR], dinv[:NR], gcnb[4][None])

    m5Wp = jnp.pad(m5W, ((0, 0), (0, F - 1)))
    m5bp = jnp.pad(m5b, (0, F - 1))
    ws = (m0W, m0b[None], m1W, m1b[None], m2W, m2b[None],
          m3W, m3b[None], m4W, m4b[None], m5Wp, m5bp[None])
    zz = _mlp(h64, ws)
    z = zz[:, 0].reshape(NR, NR)[:NQ, :NQ]
    return z.reshape(1, NQ * NQ)
